# Initial kernel scaffold; baseline (speedup 1.0000x reference)
import jax
import jax.numpy as jnp
from jax.experimental import pallas as pl


def kernel(pos, s, pair_rep, pair_mask, noise_level, W_rbf, W_n1, W_n2, W_mod, W_in2f, Wf1, Wf2, W_out1, W_out2, w_z):
    raise NotImplementedError("write your pallas kernel here")


# speedup vs baseline: 8.9664x; 8.9664x over previous
import jax
import jax.numpy as jnp
import numpy as np
from jax.experimental import pallas as pl
from jax.experimental.pallas import tpu as pltpu

N = 512
DIM_S = 128
DIM_Z = 64
N_RBF = 64
DIM_FILTER = 128
DIM_NOISE = 64
N_LAYERS = 3
R_MIN = 0.04
R_MAX = 10.0
EPS = 1e-5

TIA = 8
TI = 8


def _edge_kernel(pos_blk, pos_all, z_blk, pmask_blk, wrbf, wz3,
                 rbf1_out, eg_out):
    i = pl.program_id(0)
    eb = TIA * N
    pa = pos_all[...]
    pb = pos_blk[...]
    rel = pb[:, None, :] - pa[None, :, :]
    d2 = jnp.sum(rel * rel, axis=2, keepdims=True)
    d = jnp.sqrt(d2 + 1e-12).reshape(eb, 1)
    x = jnp.log(jnp.maximum(d, R_MIN))
    sigma = (np.log(R_MAX) - np.log(R_MIN)) / (N_RBF - 1)
    mu = np.log(R_MIN) + sigma * jax.lax.broadcasted_iota(
        jnp.int32, (1, N_RBF), 1).astype(jnp.float32)
    basis = jnp.exp(-0.5 * ((x - mu) / sigma) ** 2)
    fcut = jnp.exp(-0.5 * (d / (R_MAX / 3.0)) ** 2)
    basis = basis * fcut
    r1 = jnp.dot(basis, wrbf[...], preferred_element_type=jnp.float32)
    rbf1_out[...] = r1 * jax.nn.sigmoid(r1)

    jidx = jax.lax.broadcasted_iota(jnp.int32, (TIA, N, 1), 1)
    iidx = i * TIA + jax.lax.broadcasted_iota(jnp.int32, (TIA, N, 1), 0)
    mask = (jidx != iidx).astype(jnp.float32).reshape(eb, 1)
    mask = mask * pmask_blk[...]
    egs = jax.nn.sigmoid(jnp.dot(z_blk[...], wz3[...],
                                 preferred_element_type=jnp.float32))
    eg_out[...] = jnp.stack(
        [egs[:, 0:1] * mask, egs[:, 1:2] * mask, egs[:, 2:3] * mask],
        axis=0)


def _layer_kernel(rbf1, eg, s, shift, scale, gate,
                  w_in2f, wf1, wf2, wo1, wo2, out, h_buf, xf_buf):
    l = pl.program_id(0)
    i = pl.program_id(1)

    @pl.when(jnp.logical_and(l == 0, i == 0))
    def _():
        h_buf[...] = s[...]

    @pl.when(i == 0)
    def _():
        h = h_buf[...]
        mu = jnp.mean(h, axis=1, keepdims=True)
        var = jnp.mean((h - mu) ** 2, axis=1, keepdims=True)
        hn = (h - mu) / jnp.sqrt(var + EPS)
        hn = hn * (1.0 + scale[...].reshape(1, DIM_S)) + shift[...].reshape(1, DIM_S)
        xf_buf[...] = jnp.dot(hn, w_in2f[...].reshape(DIM_S, DIM_FILTER),
                              preferred_element_type=jnp.float32)

    rb = rbf1[...]
    f1 = jnp.dot(rb, wf1[...].reshape(N_RBF, DIM_FILTER),
                 preferred_element_type=jnp.float32)
    f1 = f1 * jax.nn.sigmoid(f1)
    w2 = jnp.dot(f1, wf2[...].reshape(DIM_FILTER, DIM_FILTER),
                 preferred_element_type=jnp.float32)
    m = (w2 * eg[...].reshape(TI * N, 1)).reshape(TI, N, DIM_FILTER)
    agg = jnp.sum(m * xf_buf[...][None, :, :], axis=1)
    a1 = jnp.dot(agg, wo1[...].reshape(DIM_FILTER, DIM_S),
                 preferred_element_type=jnp.float32)
    a1 = a1 * jax.nn.sigmoid(a1)
    ds = jnp.dot(a1, wo2[...].reshape(DIM_S, DIM_S),
                 preferred_element_type=jnp.float32)
    hrow = h_buf[pl.ds(i * TI, TI), :] + gate[...].reshape(1, DIM_S) * ds
    h_buf[pl.ds(i * TI, TI), :] = hrow
    out[...] = hrow


def kernel(pos, s, pair_rep, pair_mask, noise_level, W_rbf, W_n1, W_n2,
           W_mod, W_in2f, Wf1, Wf2, W_out1, W_out2, w_z):
    n = pos.shape[0]
    noise = jnp.clip(noise_level, 1e-4, 1e2)
    xn = jnp.log(noise)
    freqs = jnp.pi * (2.0 ** jnp.arange(DIM_NOISE // 2, dtype=jnp.float32))
    xf = xn[..., None] * freqs
    nemb = jnp.concatenate([jnp.sin(xf), jnp.cos(xf)], axis=-1)
    nemb = jax.nn.silu(nemb @ W_n1)
    nemb = jax.nn.silu(nemb @ W_n2)
    mods = jnp.tensordot(nemb[0], W_mod, axes=([0], [1]))
    shift = mods[:, :DIM_S].reshape(N_LAYERS, 1, DIM_S)
    scale = mods[:, DIM_S:2 * DIM_S].reshape(N_LAYERS, 1, DIM_S)
    gate = mods[:, 2 * DIM_S:].reshape(N_LAYERS, 1, DIM_S)

    pos4 = jnp.pad(pos, ((0, 0), (0, 1)))
    zf = pair_rep.reshape(n * n, DIM_Z)
    pm = pair_mask.reshape(n * n, 1)
    wz3 = jnp.transpose(w_z[:, :, 0])

    nba = n // TIA
    rbf1, eg = pl.pallas_call(
        _edge_kernel,
        grid=(nba,),
        in_specs=[
            pl.BlockSpec((TIA, 4), lambda i: (i, 0)),
            pl.BlockSpec((n, 4), lambda i: (0, 0)),
            pl.BlockSpec((TIA * n, DIM_Z), lambda i: (i, 0)),
            pl.BlockSpec((TIA * n, 1), lambda i: (i, 0)),
            pl.BlockSpec((N_RBF, N_RBF), lambda i: (0, 0)),
            pl.BlockSpec((DIM_Z, N_LAYERS), lambda i: (0, 0)),
        ],
        out_specs=[
            pl.BlockSpec((TIA * n, N_RBF), lambda i: (i, 0)),
            pl.BlockSpec((N_LAYERS, TIA * n, 1), lambda i: (0, i, 0)),
        ],
        out_shape=[
            jax.ShapeDtypeStruct((n * n, N_RBF), jnp.float32),
            jax.ShapeDtypeStruct((N_LAYERS, n * n, 1), jnp.float32),
        ],
    )(pos4, pos4, zf, pm, W_rbf, wz3)

    nb = n // TI
    h = pl.pallas_call(
        _layer_kernel,
        grid=(N_LAYERS, nb),
        in_specs=[
            pl.BlockSpec((TI * n, N_RBF), lambda l, i: (i, 0)),
            pl.BlockSpec((1, TI * n, 1), lambda l, i: (l, i, 0)),
            pl.BlockSpec((n, DIM_S), lambda l, i: (0, 0)),
            pl.BlockSpec((1, 1, DIM_S), lambda l, i: (l, 0, 0)),
            pl.BlockSpec((1, 1, DIM_S), lambda l, i: (l, 0, 0)),
            pl.BlockSpec((1, 1, DIM_S), lambda l, i: (l, 0, 0)),
            pl.BlockSpec((1, DIM_S, DIM_FILTER), lambda l, i: (l, 0, 0)),
            pl.BlockSpec((1, N_RBF, DIM_FILTER), lambda l, i: (l, 0, 0)),
            pl.BlockSpec((1, DIM_FILTER, DIM_FILTER), lambda l, i: (l, 0, 0)),
            pl.BlockSpec((1, DIM_FILTER, DIM_S), lambda l, i: (l, 0, 0)),
            pl.BlockSpec((1, DIM_S, DIM_S), lambda l, i: (l, 0, 0)),
        ],
        out_specs=pl.BlockSpec((TI, DIM_S), lambda l, i: (i, 0)),
        out_shape=jax.ShapeDtypeStruct((n, DIM_S), jnp.float32),
        scratch_shapes=[
            pltpu.VMEM((n, DIM_S), jnp.float32),
            pltpu.VMEM((n, DIM_S), jnp.float32),
        ],
    )(rbf1, eg, s, shift, scale, gate, W_in2f, Wf1, Wf2, W_out1, W_out2)
    return h
